# baseline (speedup 1.0000x reference)
import jax
import jax.numpy as jnp
from jax import lax
from jax.experimental import pallas as pl
from jax.experimental.pallas import tpu as pltpu

_U = 1024
_S = _U + 1


def _round_up(x, m):
    return (x + m - 1) // m * m


def _gather_segsum_kernel(a2_ref,
                          p_ref,
                          feat2_ref,
                          out_ref,
                          tile_ref):
    k = pl.program_id(0)
    nk = pl.num_programs(0)
    base = k * _U

    for mi in range(_U):
        a2 = pl.multiple_of(a2_ref[base + mi], 2)
        slab = feat2_ref[pl.ds(a2, 2), :]
        tile_ref[mi:mi + 2 * _S:_S, :] = slab

    x = jnp.concatenate(
        [tile_ref[pl.ds(0, _U), :], tile_ref[pl.ds(_S, _U), :]], axis=-1)

    P = out_ref.shape[0]
    p_row = p_ref[pl.ds(k, 1)][0]
    memb = jnp.where(
        lax.broadcasted_iota(jnp.int32, (P, _U), 0) == p_row, 1.0, 0.0)

    @pl.when(k == 0)
    def _():
        out_ref[...] = jnp.zeros(out_ref.shape, jnp.float32)

    out_ref[...] += jnp.dot(memb, x, preferred_element_type=jnp.float32)

    @pl.when(k == nk - 1)
    def _():
        portf = out_ref[...]
        sumsq = jnp.sum(portf * portf, axis=-1, keepdims=True)
        out_ref[...] = portf * lax.rsqrt(sumsq + 1e-24)


def _scores_kernel(portf_ref,
                   targ_ref,
                   out_ref):
    out_ref[...] = jnp.dot(portf_ref[...], targ_ref[...],
                           preferred_element_type=jnp.float32)


def kernel(asset_indices, portfolio_indices, feat_table, target_table_t):
    N = int(asset_indices.shape[0])
    A, D = feat_table.shape
    P = 256
    P_pad = _round_up(max(P, 8), 8)
    n_chunks = N // _U

    a2 = asset_indices.astype(jnp.int32) * 2
    p_idx = portfolio_indices.astype(jnp.int32).reshape(n_chunks, 1, _U)
    feat2 = feat_table.reshape(2 * A, 128)

    portf = pl.pallas_call(
        _gather_segsum_kernel,
        out_shape=jax.ShapeDtypeStruct((P_pad, D), jnp.float32),
        grid_spec=pltpu.PrefetchScalarGridSpec(
            num_scalar_prefetch=1,
            grid=(n_chunks,),
            in_specs=[
                pl.BlockSpec((n_chunks, 1, _U), lambda k, a_sc: (0, 0, 0)),
                pl.BlockSpec((2 * A, 128), lambda k, a_sc: (0, 0)),
            ],
            out_specs=pl.BlockSpec((P_pad, D), lambda k, a_sc: (0, 0)),
            scratch_shapes=[
                pltpu.VMEM((2 * _S, 128), jnp.float32),
            ],
        ),
        compiler_params=pltpu.CompilerParams(
            dimension_semantics=("arbitrary",),
            vmem_limit_bytes=56 << 20,
        ),
    )(a2, p_idx, feat2)

    tile_a = min(4096, A)
    grid_a = A // tile_a
    scores = pl.pallas_call(
        _scores_kernel,
        out_shape=jax.ShapeDtypeStruct((P_pad, A), jnp.float32),
        grid=(grid_a,),
        in_specs=[
            pl.BlockSpec((P_pad, D), lambda j: (0, 0)),
            pl.BlockSpec((D, tile_a), lambda j: (0, j)),
        ],
        out_specs=pl.BlockSpec((P_pad, tile_a), lambda j: (0, j)),
        compiler_params=pltpu.CompilerParams(
            dimension_semantics=("arbitrary",),
            vmem_limit_bytes=48 << 20,
        ),
    )(portf, target_table_t)

    return scores[:P]


# speedup vs baseline: 1.0923x; 1.0923x over previous
import jax
import jax.numpy as jnp
from jax import lax
from jax.experimental import pallas as pl
from jax.experimental.pallas import tpu as pltpu

_U = 1024
_S = _U + 1
_TILE_A = 4096


def _round_up(x, m):
    return (x + m - 1) // m * m


def _make_fused_kernel(NK, NT, P, D):
    def fused_kernel(a2_ref,
                     p_ref,
                     feat2_ref,
                     targ_hbm,
                     out_ref,
                     tile_ref,
                     portf_ref,
                     tbuf_ref,
                     sems):
        k = pl.program_id(0)

        @pl.when(k < NK)
        def _():
            base = k * _U
            for mi in range(_U):
                a2 = pl.multiple_of(a2_ref[base + mi], 2)
                slab = feat2_ref[pl.ds(a2, 2), :]
                tile_ref[mi:mi + 2 * _S:_S, :] = slab

            x = jnp.concatenate(
                [tile_ref[pl.ds(0, _U), :], tile_ref[pl.ds(_S, _U), :]],
                axis=-1)

            p_row = p_ref[pl.ds(k, 1)][0]
            memb = jnp.where(
                lax.broadcasted_iota(jnp.int32, (P, _U), 0) == p_row,
                1.0, 0.0)

            @pl.when(k == 0)
            def _():
                portf_ref[...] = jnp.zeros((P, D), jnp.float32)

            portf_ref[...] += jnp.dot(memb, x,
                                      preferred_element_type=jnp.float32)

            @pl.when(k == NK - 1)
            def _():
                portf = portf_ref[...]
                sumsq = jnp.sum(portf * portf, axis=-1, keepdims=True)
                portf_ref[...] = portf * lax.rsqrt(sumsq + 1e-24)

        @pl.when(k < min(2, NT))
        def _():
            pltpu.make_async_copy(
                targ_hbm.at[:, pl.ds(k * _TILE_A, _TILE_A)],
                tbuf_ref.at[k],
                sems.at[k],
            ).start()

        @pl.when(k >= NK)
        def _():
            j = k - NK
            slot = lax.rem(j, 2)
            pltpu.make_async_copy(
                targ_hbm.at[:, pl.ds(j * _TILE_A, _TILE_A)],
                tbuf_ref.at[slot],
                sems.at[slot],
            ).wait()
            out_ref[...] = jnp.dot(portf_ref[...],
                                   tbuf_ref[pl.ds(slot, 1)][0],
                                   preferred_element_type=jnp.float32)

            @pl.when(j + 2 < NT)
            def _():
                pltpu.make_async_copy(
                    targ_hbm.at[:, pl.ds((j + 2) * _TILE_A, _TILE_A)],
                    tbuf_ref.at[slot],
                    sems.at[slot],
                ).start()

    return fused_kernel


def kernel(asset_indices, portfolio_indices, feat_table, target_table_t):
    N = int(asset_indices.shape[0])
    A, D = feat_table.shape
    P = 256
    P_pad = _round_up(max(P, 8), 8)
    NK = N // _U
    tile_a = min(_TILE_A, A)
    NT = A // tile_a

    a2 = asset_indices.astype(jnp.int32) * 2
    p_idx = portfolio_indices.astype(jnp.int32).reshape(NK, 1, _U)
    feat2 = feat_table.reshape(2 * A, 128)

    scores = pl.pallas_call(
        _make_fused_kernel(NK, NT, P_pad, D),
        out_shape=jax.ShapeDtypeStruct((P_pad, A), jnp.float32),
        grid_spec=pltpu.PrefetchScalarGridSpec(
            num_scalar_prefetch=1,
            grid=(NK + NT,),
            in_specs=[
                pl.BlockSpec((NK, 1, _U), lambda k, a_sc: (0, 0, 0)),
                pl.BlockSpec((2 * A, 128), lambda k, a_sc: (0, 0)),
                pl.BlockSpec(memory_space=pl.ANY),
            ],
            out_specs=pl.BlockSpec(
                (P_pad, tile_a),
                lambda k, a_sc: (0, jnp.maximum(k - NK, 0))),
            scratch_shapes=[
                pltpu.VMEM((2 * _S, 128), jnp.float32),
                pltpu.VMEM((P_pad, D), jnp.float32),
                pltpu.VMEM((2, D, tile_a), jnp.float32),
                pltpu.SemaphoreType.DMA((2,)),
            ],
        ),
        compiler_params=pltpu.CompilerParams(
            dimension_semantics=("arbitrary",),
            vmem_limit_bytes=57 << 20,
        ),
    )(a2, p_idx, feat2, target_table_t)

    return scores[:P]


# speedup vs baseline: 1.1508x; 1.0535x over previous
import jax
import jax.numpy as jnp
from jax import lax
from jax.experimental import pallas as pl
from jax.experimental.pallas import tpu as pltpu

_U = 1024
_S = _U + 1
_TILE_A = 4096


def _round_up(x, m):
    return (x + m - 1) // m * m


def _make_fused_kernel(NK, NT, P, D):
    def fused_kernel(a2_ref,
                     p_ref,
                     feat2_ref,
                     targ_hbm,
                     out_ref,
                     tile_ref,
                     portf_ref,
                     tbuf_ref,
                     sems):
        k = pl.program_id(0)

        @pl.when(k < NK)
        def _():
            base = k * _U
            for mi in range(_U):
                a2 = pl.multiple_of(a2_ref[base + mi] * 2, 2)
                slab = feat2_ref[pl.ds(a2, 2), :]
                tile_ref[mi:mi + 2 * _S:_S, :] = slab

            x = jnp.concatenate(
                [tile_ref[pl.ds(0, _U), :], tile_ref[pl.ds(_S, _U), :]],
                axis=-1)

            p_row = p_ref[pl.ds(k, 1)][0]
            memb = jnp.where(
                lax.broadcasted_iota(jnp.int32, (P, _U), 0) == p_row,
                1.0, 0.0)

            @pl.when(k == 0)
            def _():
                portf_ref[...] = jnp.zeros((P, D), jnp.float32)

            portf_ref[...] += jnp.dot(memb, x,
                                      preferred_element_type=jnp.float32)

            @pl.when(k == NK - 1)
            def _():
                portf = portf_ref[...]
                sumsq = jnp.sum(portf * portf, axis=-1, keepdims=True)
                portf_ref[...] = portf * lax.rsqrt(sumsq + 1e-24)

        @pl.when(k < min(4, NT))
        def _():
            pltpu.make_async_copy(
                targ_hbm.at[:, pl.ds(k * _TILE_A, _TILE_A)],
                tbuf_ref.at[k],
                sems.at[k],
            ).start()

        @pl.when(k >= NK)
        def _():
            j = k - NK
            slot = lax.rem(j, 4)
            pltpu.make_async_copy(
                targ_hbm.at[:, pl.ds(j * _TILE_A, _TILE_A)],
                tbuf_ref.at[slot],
                sems.at[slot],
            ).wait()
            out_ref[...] = jnp.dot(portf_ref[...],
                                   tbuf_ref[pl.ds(slot, 1)][0],
                                   preferred_element_type=jnp.float32)

            @pl.when(j + 4 < NT)
            def _():
                pltpu.make_async_copy(
                    targ_hbm.at[:, pl.ds((j + 4) * _TILE_A, _TILE_A)],
                    tbuf_ref.at[slot],
                    sems.at[slot],
                ).start()

    return fused_kernel


def kernel(asset_indices, portfolio_indices, feat_table, target_table_t):
    N = int(asset_indices.shape[0])
    A, D = feat_table.shape
    P = 256
    P_pad = _round_up(max(P, 8), 8)
    NK = N // _U
    tile_a = min(_TILE_A, A)
    NT = A // tile_a

    a2 = asset_indices.astype(jnp.int32)
    p_idx = portfolio_indices.astype(jnp.int32).reshape(NK, 1, _U)
    feat2 = feat_table.reshape(2 * A, 128)

    scores = pl.pallas_call(
        _make_fused_kernel(NK, NT, P_pad, D),
        out_shape=jax.ShapeDtypeStruct((P_pad, A), jnp.float32),
        grid_spec=pltpu.PrefetchScalarGridSpec(
            num_scalar_prefetch=1,
            grid=(NK + NT,),
            in_specs=[
                pl.BlockSpec((NK, 1, _U), lambda k, a_sc: (0, 0, 0)),
                pl.BlockSpec((2 * A, 128), lambda k, a_sc: (0, 0)),
                pl.BlockSpec(memory_space=pl.ANY),
            ],
            out_specs=pl.BlockSpec(
                (P_pad, tile_a),
                lambda k, a_sc: (0, jnp.maximum(k - NK, 0))),
            scratch_shapes=[
                pltpu.VMEM((2 * _S, 128), jnp.float32),
                pltpu.VMEM((P_pad, D), jnp.float32),
                pltpu.VMEM((4, D, tile_a), jnp.float32),
                pltpu.SemaphoreType.DMA((4,)),
            ],
        ),
        compiler_params=pltpu.CompilerParams(
            dimension_semantics=("arbitrary",),
            vmem_limit_bytes=57 << 20,
        ),
    )(a2, p_idx, feat2, target_table_t)

    return scores[:P]


# speedup vs baseline: 1.1660x; 1.0133x over previous
import jax
import jax.numpy as jnp
from jax import lax
from jax.experimental import pallas as pl
from jax.experimental.pallas import tpu as pltpu

_U = 2048
_S = _U + 1
_TILE_A = 4096


def _round_up(x, m):
    return (x + m - 1) // m * m


def _make_fused_kernel(NK, NT, P, D):
    def fused_kernel(a2_ref,
                     p_ref,
                     feat2_ref,
                     targ_hbm,
                     out_ref,
                     tile_ref,
                     portf_ref,
                     tbuf_ref,
                     sems):
        k = pl.program_id(0)

        @pl.when(k < NK)
        def _():
            base = k * _U
            for mi in range(_U):
                a2 = pl.multiple_of(a2_ref[base + mi] * 2, 2)
                slab = feat2_ref[pl.ds(a2, 2), :]
                tile_ref[mi:mi + 2 * _S:_S, :] = slab

            x = jnp.concatenate(
                [tile_ref[pl.ds(0, _U), :], tile_ref[pl.ds(_S, _U), :]],
                axis=-1)

            p_row = p_ref[pl.ds(k, 1)][0]
            memb = jnp.where(
                lax.broadcasted_iota(jnp.int32, (P, _U), 0) == p_row,
                1.0, 0.0)

            @pl.when(k == 0)
            def _():
                portf_ref[...] = jnp.zeros((P, D), jnp.float32)

            portf_ref[...] += jnp.dot(memb, x,
                                      preferred_element_type=jnp.float32)

            @pl.when(k == NK - 1)
            def _():
                portf = portf_ref[...]
                sumsq = jnp.sum(portf * portf, axis=-1, keepdims=True)
                portf_ref[...] = portf * lax.rsqrt(sumsq + 1e-24)

        @pl.when(k < min(4, NT))
        def _():
            pltpu.make_async_copy(
                targ_hbm.at[:, pl.ds(k * _TILE_A, _TILE_A)],
                tbuf_ref.at[k],
                sems.at[k],
            ).start()

        @pl.when(k >= NK)
        def _():
            j = k - NK
            slot = lax.rem(j, 4)
            pltpu.make_async_copy(
                targ_hbm.at[:, pl.ds(j * _TILE_A, _TILE_A)],
                tbuf_ref.at[slot],
                sems.at[slot],
            ).wait()
            out_ref[...] = jnp.dot(portf_ref[...],
                                   tbuf_ref[pl.ds(slot, 1)][0],
                                   preferred_element_type=jnp.float32)

            @pl.when(j + 4 < NT)
            def _():
                pltpu.make_async_copy(
                    targ_hbm.at[:, pl.ds((j + 4) * _TILE_A, _TILE_A)],
                    tbuf_ref.at[slot],
                    sems.at[slot],
                ).start()

    return fused_kernel


def kernel(asset_indices, portfolio_indices, feat_table, target_table_t):
    N = int(asset_indices.shape[0])
    A, D = feat_table.shape
    P = 256
    P_pad = _round_up(max(P, 8), 8)
    NK = N // _U
    tile_a = min(_TILE_A, A)
    NT = A // tile_a

    a2 = asset_indices.astype(jnp.int32)
    p_idx = portfolio_indices.astype(jnp.int32).reshape(NK, 1, _U)
    feat2 = feat_table.reshape(2 * A, 128)

    scores = pl.pallas_call(
        _make_fused_kernel(NK, NT, P_pad, D),
        out_shape=jax.ShapeDtypeStruct((P_pad, A), jnp.float32),
        grid_spec=pltpu.PrefetchScalarGridSpec(
            num_scalar_prefetch=1,
            grid=(NK + NT,),
            in_specs=[
                pl.BlockSpec((NK, 1, _U), lambda k, a_sc: (0, 0, 0)),
                pl.BlockSpec((2 * A, 128), lambda k, a_sc: (0, 0)),
                pl.BlockSpec(memory_space=pl.ANY),
            ],
            out_specs=pl.BlockSpec(
                (P_pad, tile_a),
                lambda k, a_sc: (0, jnp.maximum(k - NK, 0))),
            scratch_shapes=[
                pltpu.VMEM((2 * _S, 128), jnp.float32),
                pltpu.VMEM((P_pad, D), jnp.float32),
                pltpu.VMEM((4, D, tile_a), jnp.float32),
                pltpu.SemaphoreType.DMA((4,)),
            ],
        ),
        compiler_params=pltpu.CompilerParams(
            dimension_semantics=("arbitrary",),
            vmem_limit_bytes=57 << 20,
        ),
    )(a2, p_idx, feat2, target_table_t)

    return scores[:P]
